# speedup vs baseline: 1.0078x; 1.0078x over previous
import functools

import jax
import jax.numpy as jnp
from jax.experimental import pallas as pl

N = 10000
E = 320000
THRESH = 0.5


def _mm_norm_body(x_ref, w_ref, xn_ref, h_ref):
    x = x_ref[...]
    nrm = jnp.sqrt(jnp.sum(x * x, axis=1, keepdims=True))
    xn_ref[...] = x / jnp.maximum(nrm, 1e-12)
    h_ref[...] = jnp.dot(x, w_ref[...], preferred_element_type=jnp.float32)


def _mm_norm(x, W):
    n, f = x.shape
    fo = W.shape[1]
    bn = 2000
    grid = (n // bn,)
    return pl.pallas_call(
        _mm_norm_body,
        grid=grid,
        in_specs=[
            pl.BlockSpec((bn, f), lambda i: (i, 0)),
            pl.BlockSpec((f, fo), lambda i: (0, 0)),
        ],
        out_specs=[
            pl.BlockSpec((bn, f), lambda i: (i, 0)),
            pl.BlockSpec((bn, fo), lambda i: (i, 0)),
        ],
        out_shape=[
            jax.ShapeDtypeStruct((n, f), jnp.float32),
            jax.ShapeDtypeStruct((n, fo), jnp.float32),
        ],
    )(x, W)


def _layer(x, src, dst, W, b):
    xn, h = _mm_norm(x, W)
    sim = jnp.sum(xn[src] * xn[dst], axis=1)
    t = jnp.where((sim >= THRESH) & (src != dst), sim, 0.0)
    row_sum = jax.ops.segment_sum(t, src, num_segments=N)
    denom = row_sum[src]
    w = jnp.where(denom > 0, t / jnp.where(denom > 0, denom, 1.0), 0.0)
    deg = jax.ops.segment_sum((t > 0).astype(jnp.float32), src, num_segments=N)
    self_w = 1.0 / (deg + 1.0)
    wsum = jax.ops.segment_sum(w, dst, num_segments=N)
    dinv = 1.0 / jnp.sqrt(wsum + self_w + 1.0)
    coef = dinv[src] * w * dinv[dst]
    out = jax.ops.segment_sum(h[src] * coef[:, None], dst, num_segments=N)
    out = out + (dinv * dinv * (self_w + 1.0))[:, None] * h
    return out + b


def kernel(node_features, edge_index, edge_weight, W1, b1, W2, b2, W3, b3):
    src, dst = edge_index[0], edge_index[1]
    x = jax.nn.relu(_layer(node_features, src, dst, W1, b1))
    x = jax.nn.relu(_layer(x, src, dst, W2, b2))
    x = _layer(x, src, dst, W3, b3)
    return jax.nn.log_softmax(x, axis=1)


# speedup vs baseline: 4.9476x; 4.9091x over previous
import functools

import jax
import jax.numpy as jnp
from jax import lax
from jax.experimental import pallas as pl
from jax.experimental.pallas import tpu as pltpu
from jax.experimental.pallas import tpu_sc as plsc

N = 10000
E = 320000
THRESH = 0.5

NC = 2
NS = 16
NW = NC * NS
NP = 10240
EW = E // NW
CH = 80
CHV = CH // 16
NB = NP // NS
NH = NP // NC
AR = NH + 256
AT = AR // NS
ZR = 48
NT = NH // NS
ER = 64
FH = 128


def _mesh():
    return plsc.VectorSubcoreMesh(
        core_axis_name="c", subcore_axis_name="s",
        num_cores=NC, num_subcores=NS)


def _tc1_body(x_ref, w_ref, xn_ref, h0_ref, h1_ref):
    x = x_ref[...]
    nrm = jnp.sqrt(jnp.sum(x * x, axis=1, keepdims=True))
    xn_ref[...] = x / jnp.maximum(nrm, 1e-12)
    h = jnp.dot(x, w_ref[...], preferred_element_type=jnp.float32)
    h0_ref[...] = h[:, :128]
    h1_ref[...] = h[:, 128:]


def _tc1(x, W):
    bn = 2000
    return pl.pallas_call(
        _tc1_body,
        grid=(N // bn,),
        in_specs=[
            pl.BlockSpec((bn, 128), lambda i: (i, 0)),
            pl.BlockSpec((128, 256), lambda i: (0, 0)),
        ],
        out_specs=[
            pl.BlockSpec((bn, 128), lambda i: (i, 0)),
            pl.BlockSpec((bn, 128), lambda i: (i, 0)),
            pl.BlockSpec((bn, 128), lambda i: (i, 0)),
        ],
        out_shape=[
            jax.ShapeDtypeStruct((N, 128), jnp.float32),
            jax.ShapeDtypeStruct((N, 128), jnp.float32),
            jax.ShapeDtypeStruct((N, 128), jnp.float32),
        ],
    )(x, W)


def _tc2_body(ya, yb, b_ref, w_ref, xn_ref, h_ref):
    xa = jnp.maximum(ya[...] + b_ref[0:1, :128], 0.0)
    xb = jnp.maximum(yb[...] + b_ref[0:1, 128:], 0.0)
    x = jnp.concatenate([xa, xb], axis=1)
    nrm = jnp.sqrt(jnp.sum(x * x, axis=1, keepdims=True))
    xn_ref[...] = x / jnp.maximum(nrm, 1e-12)
    h = jnp.dot(x, w_ref[...], preferred_element_type=jnp.float32)
    h_ref[...] = jnp.concatenate(
        [h, jnp.zeros((h.shape[0], 128 - h.shape[1]), jnp.float32)], axis=1)


def _tc2(ya, yb, b1, W2):
    bn = 2000
    return pl.pallas_call(
        _tc2_body,
        grid=(N // bn,),
        in_specs=[
            pl.BlockSpec((bn, 128), lambda i: (i, 0)),
            pl.BlockSpec((bn, 128), lambda i: (i, 0)),
            pl.BlockSpec((1, 256), lambda i: (0, 0)),
            pl.BlockSpec((256, 16), lambda i: (0, 0)),
        ],
        out_specs=[
            pl.BlockSpec((bn, 256), lambda i: (i, 0)),
            pl.BlockSpec((bn, 128), lambda i: (i, 0)),
        ],
        out_shape=[
            jax.ShapeDtypeStruct((N, 256), jnp.float32),
            jax.ShapeDtypeStruct((N, 128), jnp.float32),
        ],
    )(ya, yb, b1.reshape(1, 256), W2)


def _tc3_body(y_ref, b_ref, w_ref, xn_ref, h_ref):
    x = jnp.maximum(y_ref[:, :16] + b_ref[0:1, :], 0.0)
    nrm = jnp.sqrt(jnp.sum(x * x, axis=1, keepdims=True))
    xn = x / jnp.maximum(nrm, 1e-12)
    pad = jnp.zeros((x.shape[0], 112), jnp.float32)
    xn_ref[...] = jnp.concatenate([xn, pad], axis=1)
    h_ref[...] = jnp.dot(x, w_ref[...], preferred_element_type=jnp.float32)


def _tc3(y2, b2, W3p):
    bn = 2000
    return pl.pallas_call(
        _tc3_body,
        grid=(N // bn,),
        in_specs=[
            pl.BlockSpec((bn, 128), lambda i: (i, 0)),
            pl.BlockSpec((1, 16), lambda i: (0, 0)),
            pl.BlockSpec((16, 128), lambda i: (0, 0)),
        ],
        out_specs=[
            pl.BlockSpec((bn, 128), lambda i: (i, 0)),
            pl.BlockSpec((bn, 128), lambda i: (i, 0)),
        ],
        out_shape=[
            jax.ShapeDtypeStruct((N, 128), jnp.float32),
            jax.ShapeDtypeStruct((N, 128), jnp.float32),
        ],
    )(y2, b2.reshape(1, 16), W3p)


def _tc_node_body(cnt_ref, ws_ref, dinv_ref, selfco_ref):
    cnt = cnt_ref[0] + cnt_ref[1]
    ws = ws_ref[0] + ws_ref[1]
    sw = 1.0 / (cnt + 1.0)
    dg = ws + sw + 1.0
    dinv = 1.0 / jnp.sqrt(dg)
    dinv_ref[...] = dinv
    selfco_ref[...] = dinv * dinv * (sw + 1.0)


def _tc_node(cnt_part, ws_part):
    c2 = cnt_part.reshape(NC, 80, 128)
    w2 = ws_part.reshape(NC, 80, 128)
    dinv, selfco = pl.pallas_call(
        _tc_node_body,
        grid=(1,),
        in_specs=[
            pl.BlockSpec((NC, 80, 128), lambda i: (0, 0, 0)),
            pl.BlockSpec((NC, 80, 128), lambda i: (0, 0, 0)),
        ],
        out_specs=[
            pl.BlockSpec((80, 128), lambda i: (0, 0)),
            pl.BlockSpec((80, 128), lambda i: (0, 0)),
        ],
        out_shape=[
            jax.ShapeDtypeStruct((80, 128), jnp.float32),
            jax.ShapeDtypeStruct((80, 128), jnp.float32),
        ],
    )(c2, w2)
    return dinv.reshape(NP), selfco.reshape(NP)


def _tc4_body(y_ref, b_ref, o_ref):
    xx = y_ref[:, :40] + b_ref[0:1, :]
    m = jnp.max(xx, axis=1, keepdims=True)
    e = jnp.exp(xx - m)
    o_ref[...] = xx - m - jnp.log(jnp.sum(e, axis=1, keepdims=True))


def _tc4(y3, b3):
    bn = 2000
    return pl.pallas_call(
        _tc4_body,
        grid=(N // bn,),
        in_specs=[
            pl.BlockSpec((bn, 128), lambda i: (i, 0)),
            pl.BlockSpec((1, 40), lambda i: (0, 0)),
        ],
        out_specs=pl.BlockSpec((bn, 40), lambda i: (i, 0)),
        out_shape=jax.ShapeDtypeStruct((N, 40), jnp.float32),
    )(y3, b3.reshape(1, 40))


def _make_p1(F):
    mesh = _mesh()

    @functools.partial(
        pl.kernel, mesh=mesh,
        out_type=[
            jax.ShapeDtypeStruct((E,), jnp.float32),
            jax.ShapeDtypeStruct((NC * NP,), jnp.float32),
            jax.ShapeDtypeStruct((NC * NP,), jnp.float32),
        ],
        scratch_types=[
            pltpu.VMEM((CH,), jnp.int32),
            pltpu.VMEM((CH,), jnp.int32),
            pltpu.VMEM((CH, F), jnp.float32),
            pltpu.VMEM((CH, F), jnp.float32),
            pltpu.VMEM((CH,), jnp.float32),
            pltpu.VMEM((CH,), jnp.float32),
            pltpu.VMEM((256,), jnp.float32),
            pltpu.VMEM((NB,), jnp.float32),
            pltpu.VMEM_SHARED((NP,), jnp.float32),
            pltpu.VMEM_SHARED((NP,), jnp.float32),
            pltpu.SemaphoreType.DMA,
            pltpu.SemaphoreType.DMA,
        ],
        compiler_params=pltpu.CompilerParams(needs_layout_passes=False),
        name=f"gnn_p1_f{F}",
    )
    def p1(xn_hbm, src_hbm, dst_hbm, t_hbm, rs_hbm, cnt_hbm,
           src_v, dst_v, xs_v, xd_v, t_v, cnt_v, red_v, zb, acc_rs, acc_cnt,
           sem1, sem2):
        cid = lax.axis_index("c")
        sid = lax.axis_index("s")
        wid = cid * NS + sid
        zero = jnp.zeros((16,), jnp.float32)

        def zj(j, carry):
            zb[pl.ds(j * 16, 16)] = zero
            return carry

        lax.fori_loop(0, NB // 16, zj, 0)
        pltpu.sync_copy(zb, acc_rs.at[pl.ds(sid * NB, NB)])
        pltpu.sync_copy(zb, acc_cnt.at[pl.ds(sid * NB, NB)])
        plsc.subcore_barrier()

        def chunk(k, carry):
            base = wid * EW + k * CH
            pltpu.sync_copy(src_hbm.at[pl.ds(base, CH)], src_v)
            pltpu.sync_copy(dst_hbm.at[pl.ds(base, CH)], dst_v)
            ca = pltpu.async_copy(xn_hbm.at[src_v], xs_v, sem1)
            cb = pltpu.async_copy(xn_hbm.at[dst_v], xd_v, sem2)
            ca.wait()
            cb.wait()

            lanes16 = lax.iota(jnp.int32, 16) * 16

            def grp(i, gcarry):
                for l in range(16):
                    e = i * 16 + l
                    acc = xs_v[e, pl.ds(0, 16)] * xd_v[e, pl.ds(0, 16)]
                    for f in range(1, F // 16):
                        sl = pl.ds(f * 16, 16)
                        acc = acc + xs_v[e, sl] * xd_v[e, sl]
                    red_v[pl.ds(l * 16, 16)] = acc
                raw = plsc.load_gather(red_v, [lanes16])
                for j in range(1, 16):
                    raw = raw + plsc.load_gather(red_v, [lanes16 + j])
                sl = pl.ds(i * 16, 16)
                s16 = src_v[sl]
                d16 = dst_v[sl]
                t16 = jnp.where((raw >= THRESH) & (s16 != d16), raw, 0.0)
                t_v[sl] = t16
                cnt_v[sl] = jnp.where(t16 > 0.0, 1.0, 0.0)
                return gcarry

            lax.fori_loop(0, CHV, grp, 0)
            pltpu.sync_copy(t_v, t_hbm.at[pl.ds(base, CH)])
            pltpu.sync_copy(t_v, acc_rs.at[src_v], add=True)
            pltpu.sync_copy(cnt_v, acc_cnt.at[src_v], add=True)
            return carry

        lax.fori_loop(0, EW // CH, chunk, 0)
        plsc.subcore_barrier()
        sl = pl.ds(sid * NB, NB)
        osl = pl.ds(cid * NP + sid * NB, NB)
        pltpu.sync_copy(acc_rs.at[sl], rs_hbm.at[osl])
        pltpu.sync_copy(acc_cnt.at[sl], cnt_hbm.at[osl])

    return p1


def _make_p2():
    mesh = _mesh()

    @functools.partial(
        pl.kernel, mesh=mesh,
        out_type=[
            jax.ShapeDtypeStruct((E,), jnp.float32),
            jax.ShapeDtypeStruct((NC * NP,), jnp.float32),
        ],
        scratch_types=[
            pltpu.VMEM((CH,), jnp.int32),
            pltpu.VMEM((CH,), jnp.int32),
            pltpu.VMEM((CH,), jnp.float32),
            pltpu.VMEM((CH,), jnp.float32),
            pltpu.VMEM((NP,), jnp.float32),
            pltpu.VMEM((NP,), jnp.float32),
            pltpu.VMEM((NB,), jnp.float32),
            pltpu.VMEM_SHARED((NP,), jnp.float32),
        ],
        compiler_params=pltpu.CompilerParams(needs_layout_passes=False),
        name="gnn_p2",
    )
    def p2(t_hbm, src_hbm, dst_hbm, rs_hbm, w_hbm, ws_hbm,
           src_v, dst_v, t_v, w_v, rs_full, tmp, zb, acc_ws):
        cid = lax.axis_index("c")
        sid = lax.axis_index("s")
        wid = cid * NS + sid
        zero = jnp.zeros((16,), jnp.float32)
        pltpu.sync_copy(rs_hbm.at[pl.ds(0, NP)], rs_full)
        pltpu.sync_copy(rs_hbm.at[pl.ds(NP, NP)], tmp)

        def addj(j, carry):
            sj = pl.ds(j * 16, 16)
            rs_full[sj] = rs_full[sj] + tmp[sj]
            return carry

        lax.fori_loop(0, NP // 16, addj, 0)

        def zj(j, carry):
            zb[pl.ds(j * 16, 16)] = zero
            return carry

        lax.fori_loop(0, NB // 16, zj, 0)
        pltpu.sync_copy(zb, acc_ws.at[pl.ds(sid * NB, NB)])
        plsc.subcore_barrier()

        def chunk(k, carry):
            base = wid * EW + k * CH
            pltpu.sync_copy(src_hbm.at[pl.ds(base, CH)], src_v)
            pltpu.sync_copy(dst_hbm.at[pl.ds(base, CH)], dst_v)
            pltpu.sync_copy(t_hbm.at[pl.ds(base, CH)], t_v)

            def grp(i, gcarry):
                sl = pl.ds(i * 16, 16)
                s16 = src_v[sl]
                rs16 = plsc.load_gather(rs_full, [s16])
                t16 = t_v[sl]
                w_v[sl] = jnp.where(rs16 > 0.0, t16 / rs16, 0.0)
                return gcarry

            lax.fori_loop(0, CHV, grp, 0)
            pltpu.sync_copy(w_v, w_hbm.at[pl.ds(base, CH)])
            pltpu.sync_copy(w_v, acc_ws.at[dst_v], add=True)
            return carry

        lax.fori_loop(0, EW // CH, chunk, 0)
        plsc.subcore_barrier()
        sl = pl.ds(sid * NB, NB)
        pltpu.sync_copy(acc_ws.at[sl],
                        ws_hbm.at[pl.ds(cid * NP + sid * NB, NB)])

    return p2


def _make_p3():
    mesh = _mesh()

    @functools.partial(
        pl.kernel, mesh=mesh,
        out_type=jax.ShapeDtypeStruct((NP, FH), jnp.float32),
        scratch_types=[
            pltpu.VMEM((CH,), jnp.int32),
            pltpu.VMEM((CH,), jnp.int32),
            pltpu.VMEM((CH,), jnp.float32),
            pltpu.VMEM((CH, FH), jnp.float32),
            pltpu.VMEM((NP,), jnp.float32),
            pltpu.VMEM((NP,), jnp.float32),
            pltpu.VMEM((ZR, FH), jnp.float32),
            pltpu.VMEM((ER, FH), jnp.float32),
            pltpu.VMEM((ER, FH), jnp.float32),
            pltpu.VMEM_SHARED((AR, FH), jnp.float32),
            pltpu.SemaphoreType.DMA,
        ],
        compiler_params=pltpu.CompilerParams(needs_layout_passes=False),
        name="gnn_p3",
    )
    def p3(h_hbm, w_hbm, src_hbm, dst_hbm, dinv_hbm, selfco_hbm, out_hbm,
           src_v, dst_v, c_v, hrow_v, dinv_t, selfco_t, zbuf, ebuf, hbuf,
           acc, sem1):
        cid = lax.axis_index("c")
        sid = lax.axis_index("s")
        lo = cid * NH
        zero16 = jnp.zeros((16,), jnp.float32)

        pltpu.sync_copy(dinv_hbm, dinv_t)
        pltpu.sync_copy(selfco_hbm, selfco_t)

        def zrow(r, carry):
            for f in range(FH // 16):
                zbuf[r, pl.ds(f * 16, 16)] = zero16
            return carry

        lax.fori_loop(0, ZR, zrow, 0)
        for z in range(AT // ZR):
            pltpu.sync_copy(zbuf, acc.at[pl.ds(sid * AT + z * ZR, ZR)])
        plsc.subcore_barrier()

        def chunk(k, carry):
            base = sid * NC * EW + k * CH
            pltpu.sync_copy(src_hbm.at[pl.ds(base, CH)], src_v)
            pltpu.sync_copy(dst_hbm.at[pl.ds(base, CH)], dst_v)
            pltpu.sync_copy(w_hbm.at[pl.ds(base, CH)], c_v)
            ca = pltpu.async_copy(h_hbm.at[src_v], hrow_v, sem1)
            ca.wait()

            def grp(i, gcarry):
                sl = pl.ds(i * 16, 16)
                s16 = src_v[sl]
                d16 = dst_v[sl]
                ds16 = plsc.load_gather(dinv_t, [s16])
                dd16 = plsc.load_gather(dinv_t, [d16])
                c_v[sl] = c_v[sl] * ds16 * dd16
                dl16 = d16 - lo
                mine = (dl16 >= 0) & (dl16 < NH)
                dst_v[sl] = jnp.where(mine, dl16, NH)
                return gcarry

            lax.fori_loop(0, CHV, grp, 0)

            def edge(e, ecarry):
                cb = plsc.load_gather(c_v, [jnp.full((16,), e, jnp.int32)])
                for f in range(FH // 16):
                    sl = pl.ds(f * 16, 16)
                    hrow_v[e, sl] = hrow_v[e, sl] * cb
                return ecarry

            lax.fori_loop(0, CH, edge, 0)
            pltpu.sync_copy(hrow_v, acc.at[dst_v], add=True)
            return carry

        lax.fori_loop(0, NC * EW // CH, chunk, 0)
        plsc.subcore_barrier()

        for chk in range(NT // ER):
            n0l = sid * NT + chk * ER
            n0g = lo + n0l
            pltpu.sync_copy(acc.at[pl.ds(n0l, ER)], ebuf)
            pltpu.sync_copy(h_hbm.at[pl.ds(n0g, ER)], hbuf)

            def srow(r, carry):
                cb = plsc.load_gather(
                    selfco_t, [jnp.full((16,), n0g + r, jnp.int32)])
                for f in range(FH // 16):
                    sl = pl.ds(f * 16, 16)
                    ebuf[r, sl] = ebuf[r, sl] + cb * hbuf[r, sl]
                return carry

            lax.fori_loop(0, ER, srow, 0)
            pltpu.sync_copy(ebuf, out_hbm.at[pl.ds(n0g, ER)])

    return p3


_P1_128 = _make_p1(128)
_P1_256 = _make_p1(256)
_P2 = _make_p2()
_P3 = _make_p3()


def _layer_sparse(xn, h_parts, src, dst, p1):
    t, rs, cnt = p1(xn, src, dst)
    w, ws = _P2(t, src, dst, rs)
    dinv, selfco = _tc_node(cnt, ws)
    out = []
    for h in h_parts:
        hp = jnp.pad(h, ((0, NP - N), (0, 0)))
        out.append(_P3(hp, w, src, dst, dinv, selfco)[:N])
    return out


def kernel(node_features, edge_index, edge_weight, W1, b1, W2, b2, W3, b3):
    src = edge_index[0]
    dst = edge_index[1]

    xn0, h0, h1 = _tc1(node_features, W1)
    y1a, y1b = _layer_sparse(xn0, [h0, h1], src, dst, _P1_128)

    xn1, h2 = _tc2(y1a, y1b, b1, W2)
    (y2,) = _layer_sparse(xn1, [h2], src, dst, _P1_256)

    W3p = jnp.pad(W3, ((0, 0), (0, 88)))
    xn2, h3 = _tc3(y2, b2, W3p)
    (y3,) = _layer_sparse(xn2, [h3], src, dst, _P1_128)

    return _tc4(y3, b3)


# speedup vs baseline: 7.7536x; 1.5672x over previous
import functools

import jax
import jax.numpy as jnp
from jax import lax
from jax.experimental import pallas as pl
from jax.experimental.pallas import tpu as pltpu
from jax.experimental.pallas import tpu_sc as plsc

N = 10000
E = 320000
THRESH = 0.5

NC = 2
NS = 16
NW = NC * NS
NP = 10240
EW = E // NW
CH = 80
CHV = CH // 16
NB = NP // NS
NH = NP // NC
AR = NH + 256
AT = AR // NS
ZR = 48
NT = NH // NS
ER = 64
FH = 128


def _mesh():
    return plsc.VectorSubcoreMesh(
        core_axis_name="c", subcore_axis_name="s",
        num_cores=NC, num_subcores=NS)


def _tc1_body(x_ref, w_ref, xn_ref, h0_ref, h1_ref):
    x = x_ref[...]
    nrm = jnp.sqrt(jnp.sum(x * x, axis=1, keepdims=True))
    xn_ref[...] = x / jnp.maximum(nrm, 1e-12)
    h = jnp.dot(x, w_ref[...], preferred_element_type=jnp.float32)
    h0_ref[...] = h[:, :128]
    h1_ref[...] = h[:, 128:]


def _tc1(x, W):
    bn = 2000
    return pl.pallas_call(
        _tc1_body,
        grid=(N // bn,),
        in_specs=[
            pl.BlockSpec((bn, 128), lambda i: (i, 0)),
            pl.BlockSpec((128, 256), lambda i: (0, 0)),
        ],
        out_specs=[
            pl.BlockSpec((bn, 128), lambda i: (i, 0)),
            pl.BlockSpec((bn, 128), lambda i: (i, 0)),
            pl.BlockSpec((bn, 128), lambda i: (i, 0)),
        ],
        out_shape=[
            jax.ShapeDtypeStruct((N, 128), jnp.float32),
            jax.ShapeDtypeStruct((N, 128), jnp.float32),
            jax.ShapeDtypeStruct((N, 128), jnp.float32),
        ],
    )(x, W)


def _tc2_body(ya, yb, b_ref, w_ref, xn_ref, h_ref):
    xa = jnp.maximum(ya[...] + b_ref[0:1, :128], 0.0)
    xb = jnp.maximum(yb[...] + b_ref[0:1, 128:], 0.0)
    x = jnp.concatenate([xa, xb], axis=1)
    nrm = jnp.sqrt(jnp.sum(x * x, axis=1, keepdims=True))
    xn_ref[...] = x / jnp.maximum(nrm, 1e-12)
    h = jnp.dot(x, w_ref[...], preferred_element_type=jnp.float32)
    h_ref[...] = jnp.concatenate(
        [h, jnp.zeros((h.shape[0], 128 - h.shape[1]), jnp.float32)], axis=1)


def _tc2(ya, yb, b1, W2):
    bn = 2000
    return pl.pallas_call(
        _tc2_body,
        grid=(N // bn,),
        in_specs=[
            pl.BlockSpec((bn, 128), lambda i: (i, 0)),
            pl.BlockSpec((bn, 128), lambda i: (i, 0)),
            pl.BlockSpec((1, 256), lambda i: (0, 0)),
            pl.BlockSpec((256, 16), lambda i: (0, 0)),
        ],
        out_specs=[
            pl.BlockSpec((bn, 256), lambda i: (i, 0)),
            pl.BlockSpec((bn, 128), lambda i: (i, 0)),
        ],
        out_shape=[
            jax.ShapeDtypeStruct((N, 256), jnp.float32),
            jax.ShapeDtypeStruct((N, 128), jnp.float32),
        ],
    )(ya, yb, b1.reshape(1, 256), W2)


def _tc3_body(y_ref, b_ref, w_ref, xn_ref, h_ref):
    x = jnp.maximum(y_ref[:, :16] + b_ref[0:1, :], 0.0)
    nrm = jnp.sqrt(jnp.sum(x * x, axis=1, keepdims=True))
    xn = x / jnp.maximum(nrm, 1e-12)
    pad = jnp.zeros((x.shape[0], 112), jnp.float32)
    xn_ref[...] = jnp.concatenate([xn, pad], axis=1)
    h_ref[...] = jnp.dot(x, w_ref[...], preferred_element_type=jnp.float32)


def _tc3(y2, b2, W3p):
    bn = 2000
    return pl.pallas_call(
        _tc3_body,
        grid=(N // bn,),
        in_specs=[
            pl.BlockSpec((bn, 128), lambda i: (i, 0)),
            pl.BlockSpec((1, 16), lambda i: (0, 0)),
            pl.BlockSpec((16, 128), lambda i: (0, 0)),
        ],
        out_specs=[
            pl.BlockSpec((bn, 128), lambda i: (i, 0)),
            pl.BlockSpec((bn, 128), lambda i: (i, 0)),
        ],
        out_shape=[
            jax.ShapeDtypeStruct((N, 128), jnp.float32),
            jax.ShapeDtypeStruct((N, 128), jnp.float32),
        ],
    )(y2, b2.reshape(1, 16), W3p)


def _tc_node_body(cnt_ref, ws_ref, dinv_ref, selfco_ref):
    cnt = cnt_ref[0] + cnt_ref[1]
    ws = ws_ref[0] + ws_ref[1]
    sw = 1.0 / (cnt + 1.0)
    dg = ws + sw + 1.0
    dinv = 1.0 / jnp.sqrt(dg)
    dinv_ref[...] = dinv
    selfco_ref[...] = dinv * dinv * (sw + 1.0)


def _tc_node(cnt_part, ws_part):
    c2 = cnt_part.reshape(NC, 80, 128)
    w2 = ws_part.reshape(NC, 80, 128)
    dinv, selfco = pl.pallas_call(
        _tc_node_body,
        grid=(1,),
        in_specs=[
            pl.BlockSpec((NC, 80, 128), lambda i: (0, 0, 0)),
            pl.BlockSpec((NC, 80, 128), lambda i: (0, 0, 0)),
        ],
        out_specs=[
            pl.BlockSpec((80, 128), lambda i: (0, 0)),
            pl.BlockSpec((80, 128), lambda i: (0, 0)),
        ],
        out_shape=[
            jax.ShapeDtypeStruct((80, 128), jnp.float32),
            jax.ShapeDtypeStruct((80, 128), jnp.float32),
        ],
    )(c2, w2)
    return dinv.reshape(NP), selfco.reshape(NP)


def _tc4_body(y_ref, b_ref, o_ref):
    xx = y_ref[:, :40] + b_ref[0:1, :]
    m = jnp.max(xx, axis=1, keepdims=True)
    e = jnp.exp(xx - m)
    o_ref[...] = xx - m - jnp.log(jnp.sum(e, axis=1, keepdims=True))


def _tc4(y3, b3):
    bn = 2000
    return pl.pallas_call(
        _tc4_body,
        grid=(N // bn,),
        in_specs=[
            pl.BlockSpec((bn, 128), lambda i: (i, 0)),
            pl.BlockSpec((1, 40), lambda i: (0, 0)),
        ],
        out_specs=pl.BlockSpec((bn, 40), lambda i: (i, 0)),
        out_shape=jax.ShapeDtypeStruct((N, 40), jnp.float32),
    )(y3, b3.reshape(1, 40))


def _make_p1(F):
    mesh = _mesh()

    @functools.partial(
        pl.kernel, mesh=mesh,
        out_type=[
            jax.ShapeDtypeStruct((E,), jnp.float32),
            jax.ShapeDtypeStruct((NC * NP,), jnp.float32),
            jax.ShapeDtypeStruct((NC * NP,), jnp.float32),
        ],
        scratch_types=[
            pltpu.VMEM((CH,), jnp.int32),
            pltpu.VMEM((CH,), jnp.int32),
            pltpu.VMEM((CH, F), jnp.float32),
            pltpu.VMEM((CH, F), jnp.float32),
            pltpu.VMEM((CH,), jnp.float32),
            pltpu.VMEM((CH,), jnp.float32),
            pltpu.VMEM((256,), jnp.float32),
            pltpu.VMEM((NB,), jnp.float32),
            pltpu.VMEM_SHARED((NP,), jnp.float32),
            pltpu.VMEM_SHARED((NP,), jnp.float32),
            pltpu.SemaphoreType.DMA,
            pltpu.SemaphoreType.DMA,
        ],
        compiler_params=pltpu.CompilerParams(needs_layout_passes=False),
        name=f"gnn_p1_f{F}",
    )
    def p1(xn_hbm, src_hbm, dst_hbm, t_hbm, rs_hbm, cnt_hbm,
           src_v, dst_v, xs_v, xd_v, t_v, cnt_v, red_v, zb, acc_rs, acc_cnt,
           sem1, sem2):
        cid = lax.axis_index("c")
        sid = lax.axis_index("s")
        wid = cid * NS + sid
        zero = jnp.zeros((16,), jnp.float32)

        def zj(j, carry):
            zb[pl.ds(j * 16, 16)] = zero
            return carry

        lax.fori_loop(0, NB // 16, zj, 0)
        pltpu.sync_copy(zb, acc_rs.at[pl.ds(sid * NB, NB)])
        pltpu.sync_copy(zb, acc_cnt.at[pl.ds(sid * NB, NB)])
        plsc.subcore_barrier()

        def chunk(k, carry):
            base = wid * EW + k * CH
            pltpu.sync_copy(src_hbm.at[pl.ds(base, CH)], src_v)
            pltpu.sync_copy(dst_hbm.at[pl.ds(base, CH)], dst_v)
            ca = pltpu.async_copy(xn_hbm.at[src_v], xs_v, sem1)
            cb = pltpu.async_copy(xn_hbm.at[dst_v], xd_v, sem2)
            ca.wait()
            cb.wait()

            lanes16 = lax.iota(jnp.int32, 16) * 16

            def grp(i, gcarry):
                for l in range(16):
                    e = i * 16 + l
                    acc = xs_v[e, pl.ds(0, 16)] * xd_v[e, pl.ds(0, 16)]
                    for f in range(1, F // 16):
                        sl = pl.ds(f * 16, 16)
                        acc = acc + xs_v[e, sl] * xd_v[e, sl]
                    red_v[pl.ds(l * 16, 16)] = acc
                raw = plsc.load_gather(red_v, [lanes16])
                for j in range(1, 16):
                    raw = raw + plsc.load_gather(red_v, [lanes16 + j])
                sl = pl.ds(i * 16, 16)
                s16 = src_v[sl]
                d16 = dst_v[sl]
                t16 = jnp.where((raw >= THRESH) & (s16 != d16), raw, 0.0)
                t_v[sl] = t16
                cnt_v[sl] = jnp.where(t16 > 0.0, 1.0, 0.0)
                return gcarry

            lax.fori_loop(0, CHV, grp, 0)
            pltpu.sync_copy(t_v, t_hbm.at[pl.ds(base, CH)])
            m = t_v[pl.ds(0, 16)]
            for i in range(1, CHV):
                m = jnp.maximum(m, t_v[pl.ds(i * 16, 16)])

            @pl.when(jnp.any(m > 0.0))
            def _do_adds():
                pltpu.sync_copy(t_v, acc_rs.at[src_v], add=True)
                pltpu.sync_copy(cnt_v, acc_cnt.at[src_v], add=True)

            return carry

        lax.fori_loop(0, EW // CH, chunk, 0)
        plsc.subcore_barrier()
        sl = pl.ds(sid * NB, NB)
        osl = pl.ds(cid * NP + sid * NB, NB)
        pltpu.sync_copy(acc_rs.at[sl], rs_hbm.at[osl])
        pltpu.sync_copy(acc_cnt.at[sl], cnt_hbm.at[osl])

    return p1


def _make_p2():
    mesh = _mesh()

    @functools.partial(
        pl.kernel, mesh=mesh,
        out_type=[
            jax.ShapeDtypeStruct((E,), jnp.float32),
            jax.ShapeDtypeStruct((NC * NP,), jnp.float32),
        ],
        scratch_types=[
            pltpu.VMEM((CH,), jnp.int32),
            pltpu.VMEM((CH,), jnp.int32),
            pltpu.VMEM((CH,), jnp.float32),
            pltpu.VMEM((CH,), jnp.float32),
            pltpu.VMEM((NP,), jnp.float32),
            pltpu.VMEM((NP,), jnp.float32),
            pltpu.VMEM((NB,), jnp.float32),
            pltpu.VMEM_SHARED((NP,), jnp.float32),
        ],
        compiler_params=pltpu.CompilerParams(needs_layout_passes=False),
        name="gnn_p2",
    )
    def p2(t_hbm, src_hbm, dst_hbm, rs_hbm, w_hbm, ws_hbm,
           src_v, dst_v, t_v, w_v, rs_full, tmp, zb, acc_ws):
        cid = lax.axis_index("c")
        sid = lax.axis_index("s")
        wid = cid * NS + sid
        zero = jnp.zeros((16,), jnp.float32)
        pltpu.sync_copy(rs_hbm.at[pl.ds(0, NP)], rs_full)
        pltpu.sync_copy(rs_hbm.at[pl.ds(NP, NP)], tmp)

        def addj(j, carry):
            sj = pl.ds(j * 16, 16)
            rs_full[sj] = rs_full[sj] + tmp[sj]
            return carry

        lax.fori_loop(0, NP // 16, addj, 0)

        def zj(j, carry):
            zb[pl.ds(j * 16, 16)] = zero
            return carry

        lax.fori_loop(0, NB // 16, zj, 0)
        pltpu.sync_copy(zb, acc_ws.at[pl.ds(sid * NB, NB)])
        plsc.subcore_barrier()

        def chunk(k, carry):
            base = wid * EW + k * CH
            pltpu.sync_copy(src_hbm.at[pl.ds(base, CH)], src_v)
            pltpu.sync_copy(dst_hbm.at[pl.ds(base, CH)], dst_v)
            pltpu.sync_copy(t_hbm.at[pl.ds(base, CH)], t_v)

            def grp(i, gcarry):
                sl = pl.ds(i * 16, 16)
                s16 = src_v[sl]
                rs16 = plsc.load_gather(rs_full, [s16])
                t16 = t_v[sl]
                w_v[sl] = jnp.where(rs16 > 0.0, t16 / rs16, 0.0)
                return gcarry

            lax.fori_loop(0, CHV, grp, 0)
            pltpu.sync_copy(w_v, w_hbm.at[pl.ds(base, CH)])
            pltpu.sync_copy(w_v, acc_ws.at[dst_v], add=True)
            return carry

        lax.fori_loop(0, EW // CH, chunk, 0)
        plsc.subcore_barrier()
        sl = pl.ds(sid * NB, NB)
        pltpu.sync_copy(acc_ws.at[sl],
                        ws_hbm.at[pl.ds(cid * NP + sid * NB, NB)])

    return p2


def _make_p3():
    mesh = _mesh()

    @functools.partial(
        pl.kernel, mesh=mesh,
        out_type=jax.ShapeDtypeStruct((NP, FH), jnp.float32),
        scratch_types=[
            pltpu.VMEM((CH,), jnp.int32),
            pltpu.VMEM((CH,), jnp.int32),
            pltpu.VMEM((CH,), jnp.float32),
            pltpu.VMEM((CH, FH), jnp.float32),
            pltpu.VMEM((NP,), jnp.float32),
            pltpu.VMEM((NP,), jnp.float32),
            pltpu.VMEM((ZR, FH), jnp.float32),
            pltpu.VMEM((ER, FH), jnp.float32),
            pltpu.VMEM((ER, FH), jnp.float32),
            pltpu.VMEM_SHARED((AR, FH), jnp.float32),
            pltpu.SemaphoreType.DMA,
        ],
        compiler_params=pltpu.CompilerParams(needs_layout_passes=False),
        name="gnn_p3",
    )
    def p3(h_hbm, w_hbm, src_hbm, dst_hbm, dinv_hbm, selfco_hbm, out_hbm,
           src_v, dst_v, c_v, hrow_v, dinv_t, selfco_t, zbuf, ebuf, hbuf,
           acc, sem1):
        cid = lax.axis_index("c")
        sid = lax.axis_index("s")
        lo = cid * NH
        zero16 = jnp.zeros((16,), jnp.float32)

        pltpu.sync_copy(dinv_hbm, dinv_t)
        pltpu.sync_copy(selfco_hbm, selfco_t)

        def zrow(r, carry):
            for f in range(FH // 16):
                zbuf[r, pl.ds(f * 16, 16)] = zero16
            return carry

        lax.fori_loop(0, ZR, zrow, 0)
        for z in range(AT // ZR):
            pltpu.sync_copy(zbuf, acc.at[pl.ds(sid * AT + z * ZR, ZR)])
        plsc.subcore_barrier()

        def chunk(k, carry):
            base = sid * NC * EW + k * CH
            pltpu.sync_copy(w_hbm.at[pl.ds(base, CH)], c_v)
            m = c_v[pl.ds(0, 16)]
            for i in range(1, CHV):
                m = jnp.maximum(m, c_v[pl.ds(i * 16, 16)])

            @pl.when(jnp.any(m > 0.0))
            def _do_chunk():
                pltpu.sync_copy(src_hbm.at[pl.ds(base, CH)], src_v)
                pltpu.sync_copy(dst_hbm.at[pl.ds(base, CH)], dst_v)
                ca = pltpu.async_copy(h_hbm.at[src_v], hrow_v, sem1)

                def grp(i, gcarry):
                    sl = pl.ds(i * 16, 16)
                    s16 = src_v[sl]
                    d16 = dst_v[sl]
                    ds16 = plsc.load_gather(dinv_t, [s16])
                    dd16 = plsc.load_gather(dinv_t, [d16])
                    c_v[sl] = c_v[sl] * ds16 * dd16
                    dl16 = d16 - lo
                    mine = (dl16 >= 0) & (dl16 < NH)
                    dst_v[sl] = jnp.where(mine, dl16, NH)
                    return gcarry

                lax.fori_loop(0, CHV, grp, 0)
                ca.wait()

                def edge(e, ecarry):
                    cb = plsc.load_gather(
                        c_v, [jnp.full((16,), e, jnp.int32)])
                    for f in range(FH // 16):
                        sl = pl.ds(f * 16, 16)
                        hrow_v[e, sl] = hrow_v[e, sl] * cb
                    return ecarry

                lax.fori_loop(0, CH, edge, 0)
                pltpu.sync_copy(hrow_v, acc.at[dst_v], add=True)

            return carry

        lax.fori_loop(0, NC * EW // CH, chunk, 0)
        plsc.subcore_barrier()

        for chk in range(NT // ER):
            n0l = sid * NT + chk * ER
            n0g = lo + n0l
            pltpu.sync_copy(acc.at[pl.ds(n0l, ER)], ebuf)
            pltpu.sync_copy(h_hbm.at[pl.ds(n0g, ER)], hbuf)

            def srow(r, carry):
                cb = plsc.load_gather(
                    selfco_t, [jnp.full((16,), n0g + r, jnp.int32)])
                for f in range(FH // 16):
                    sl = pl.ds(f * 16, 16)
                    ebuf[r, sl] = ebuf[r, sl] + cb * hbuf[r, sl]
                return carry

            lax.fori_loop(0, ER, srow, 0)
            pltpu.sync_copy(ebuf, out_hbm.at[pl.ds(n0g, ER)])

    return p3


_P1_128 = _make_p1(128)
_P1_256 = _make_p1(256)
_P2 = _make_p2()
_P3 = _make_p3()


def _layer_sparse(xn, h_parts, src, dst, p1):
    t, rs, cnt = p1(xn, src, dst)
    w, ws = _P2(t, src, dst, rs)
    dinv, selfco = _tc_node(cnt, ws)
    out = []
    for h in h_parts:
        hp = jnp.pad(h, ((0, NP - N), (0, 0)))
        out.append(_P3(hp, w, src, dst, dinv, selfco)[:N])
    return out


def kernel(node_features, edge_index, edge_weight, W1, b1, W2, b2, W3, b3):
    src = edge_index[0]
    dst = edge_index[1]

    xn0, h0, h1 = _tc1(node_features, W1)
    y1a, y1b = _layer_sparse(xn0, [h0, h1], src, dst, _P1_128)

    xn1, h2 = _tc2(y1a, y1b, b1, W2)
    (y2,) = _layer_sparse(xn1, [h2], src, dst, _P1_256)

    W3p = jnp.pad(W3, ((0, 0), (0, 88)))
    xn2, h3 = _tc3(y2, b2, W3p)
    (y3,) = _layer_sparse(xn2, [h3], src, dst, _P1_128)

    return _tc4(y3, b3)


# speedup vs baseline: 8.3667x; 1.0791x over previous
import functools

import jax
import jax.numpy as jnp
from jax import lax
from jax.experimental import pallas as pl
from jax.experimental.pallas import tpu as pltpu
from jax.experimental.pallas import tpu_sc as plsc

N = 10000
E = 320000
THRESH = 0.5

NC = 2
NS = 16
NW = NC * NS
NP = 10240
EW = E // NW
CH = 80
CHV = CH // 16
NB = NP // NS
NH = NP // NC
AR = NH + 256
AT = AR // NS
ZR = 48
NT = NH // NS
ER = 64
FH = 128


def _mesh():
    return plsc.VectorSubcoreMesh(
        core_axis_name="c", subcore_axis_name="s",
        num_cores=NC, num_subcores=NS)


def _tc1_body(x_ref, w_ref, xn_ref, h0_ref, h1_ref):
    x = x_ref[...]
    nrm = jnp.sqrt(jnp.sum(x * x, axis=1, keepdims=True))
    xn_ref[...] = x / jnp.maximum(nrm, 1e-12)
    h = jnp.dot(x, w_ref[...], preferred_element_type=jnp.float32)
    h0_ref[...] = h[:, :128]
    h1_ref[...] = h[:, 128:]


def _tc1(x, W):
    bn = 2000
    return pl.pallas_call(
        _tc1_body,
        grid=(N // bn,),
        in_specs=[
            pl.BlockSpec((bn, 128), lambda i: (i, 0)),
            pl.BlockSpec((128, 256), lambda i: (0, 0)),
        ],
        out_specs=[
            pl.BlockSpec((bn, 128), lambda i: (i, 0)),
            pl.BlockSpec((bn, 128), lambda i: (i, 0)),
            pl.BlockSpec((bn, 128), lambda i: (i, 0)),
        ],
        out_shape=[
            jax.ShapeDtypeStruct((N, 128), jnp.float32),
            jax.ShapeDtypeStruct((N, 128), jnp.float32),
            jax.ShapeDtypeStruct((N, 128), jnp.float32),
        ],
    )(x, W)


def _tc2_body(ya, yb, b_ref, w_ref, xn_ref, h_ref):
    xa = jnp.maximum(ya[...] + b_ref[0:1, :128], 0.0)
    xb = jnp.maximum(yb[...] + b_ref[0:1, 128:], 0.0)
    x = jnp.concatenate([xa, xb], axis=1)
    nrm = jnp.sqrt(jnp.sum(x * x, axis=1, keepdims=True))
    xn_ref[...] = x / jnp.maximum(nrm, 1e-12)
    h = jnp.dot(x, w_ref[...], preferred_element_type=jnp.float32)
    h_ref[...] = jnp.concatenate(
        [h, jnp.zeros((h.shape[0], 128 - h.shape[1]), jnp.float32)], axis=1)


def _tc2(ya, yb, b1, W2):
    bn = 2000
    return pl.pallas_call(
        _tc2_body,
        grid=(N // bn,),
        in_specs=[
            pl.BlockSpec((bn, 128), lambda i: (i, 0)),
            pl.BlockSpec((bn, 128), lambda i: (i, 0)),
            pl.BlockSpec((1, 256), lambda i: (0, 0)),
            pl.BlockSpec((256, 16), lambda i: (0, 0)),
        ],
        out_specs=[
            pl.BlockSpec((bn, 256), lambda i: (i, 0)),
            pl.BlockSpec((bn, 128), lambda i: (i, 0)),
        ],
        out_shape=[
            jax.ShapeDtypeStruct((N, 256), jnp.float32),
            jax.ShapeDtypeStruct((N, 128), jnp.float32),
        ],
    )(ya, yb, b1.reshape(1, 256), W2)


def _tc3_body(y_ref, b_ref, w_ref, xn_ref, h_ref):
    x = jnp.maximum(y_ref[:, :16] + b_ref[0:1, :], 0.0)
    nrm = jnp.sqrt(jnp.sum(x * x, axis=1, keepdims=True))
    xn = x / jnp.maximum(nrm, 1e-12)
    pad = jnp.zeros((x.shape[0], 112), jnp.float32)
    xn_ref[...] = jnp.concatenate([xn, pad], axis=1)
    h_ref[...] = jnp.dot(x, w_ref[...], preferred_element_type=jnp.float32)


def _tc3(y2, b2, W3p):
    bn = 2000
    return pl.pallas_call(
        _tc3_body,
        grid=(N // bn,),
        in_specs=[
            pl.BlockSpec((bn, 128), lambda i: (i, 0)),
            pl.BlockSpec((1, 16), lambda i: (0, 0)),
            pl.BlockSpec((16, 128), lambda i: (0, 0)),
        ],
        out_specs=[
            pl.BlockSpec((bn, 128), lambda i: (i, 0)),
            pl.BlockSpec((bn, 128), lambda i: (i, 0)),
        ],
        out_shape=[
            jax.ShapeDtypeStruct((N, 128), jnp.float32),
            jax.ShapeDtypeStruct((N, 128), jnp.float32),
        ],
    )(y2, b2.reshape(1, 16), W3p)


def _tc_node_body(cnt_ref, ws_ref, dinv_ref, selfco_ref):
    cnt = cnt_ref[0] + cnt_ref[1]
    ws = ws_ref[0] + ws_ref[1]
    sw = 1.0 / (cnt + 1.0)
    dg = ws + sw + 1.0
    dinv = 1.0 / jnp.sqrt(dg)
    dinv_ref[...] = dinv
    selfco_ref[...] = dinv * dinv * (sw + 1.0)


def _tc_node(cnt_part, ws_part):
    c2 = cnt_part.reshape(NC, 80, 128)
    w2 = ws_part.reshape(NC, 80, 128)
    dinv, selfco = pl.pallas_call(
        _tc_node_body,
        grid=(1,),
        in_specs=[
            pl.BlockSpec((NC, 80, 128), lambda i: (0, 0, 0)),
            pl.BlockSpec((NC, 80, 128), lambda i: (0, 0, 0)),
        ],
        out_specs=[
            pl.BlockSpec((80, 128), lambda i: (0, 0)),
            pl.BlockSpec((80, 128), lambda i: (0, 0)),
        ],
        out_shape=[
            jax.ShapeDtypeStruct((80, 128), jnp.float32),
            jax.ShapeDtypeStruct((80, 128), jnp.float32),
        ],
    )(c2, w2)
    return dinv.reshape(NP), selfco.reshape(NP)


def _tc4_body(y_ref, b_ref, o_ref):
    xx = y_ref[:, :40] + b_ref[0:1, :]
    m = jnp.max(xx, axis=1, keepdims=True)
    e = jnp.exp(xx - m)
    o_ref[...] = xx - m - jnp.log(jnp.sum(e, axis=1, keepdims=True))


def _tc4(y3, b3):
    bn = 2000
    return pl.pallas_call(
        _tc4_body,
        grid=(N // bn,),
        in_specs=[
            pl.BlockSpec((bn, 128), lambda i: (i, 0)),
            pl.BlockSpec((1, 40), lambda i: (0, 0)),
        ],
        out_specs=pl.BlockSpec((bn, 40), lambda i: (i, 0)),
        out_shape=jax.ShapeDtypeStruct((N, 40), jnp.float32),
    )(y3, b3.reshape(1, 40))


def _make_p1(F):
    mesh = _mesh()

    @functools.partial(
        pl.kernel, mesh=mesh,
        out_type=[
            jax.ShapeDtypeStruct((E,), jnp.float32),
            jax.ShapeDtypeStruct((NC * NP,), jnp.float32),
            jax.ShapeDtypeStruct((NC * NP,), jnp.float32),
        ],
        scratch_types=[
            pltpu.VMEM((CH,), jnp.int32),
            pltpu.VMEM((CH,), jnp.int32),
            pltpu.VMEM((CH, F), jnp.float32),
            pltpu.VMEM((CH, F), jnp.float32),
            pltpu.VMEM((CH,), jnp.float32),
            pltpu.VMEM((CH,), jnp.float32),
            pltpu.VMEM((256,), jnp.float32),
            pltpu.VMEM((NB,), jnp.float32),
            pltpu.VMEM_SHARED((NP,), jnp.float32),
            pltpu.VMEM_SHARED((NP,), jnp.float32),
            pltpu.SemaphoreType.DMA,
            pltpu.SemaphoreType.DMA,
        ],
        compiler_params=pltpu.CompilerParams(needs_layout_passes=False),
        name=f"gnn_p1_f{F}",
    )
    def p1(xn_hbm, src_hbm, dst_hbm, t_hbm, rs_hbm, cnt_hbm,
           src_v, dst_v, xs_v, xd_v, t_v, cnt_v, red_v, zb, acc_rs, acc_cnt,
           sem1, sem2):
        cid = lax.axis_index("c")
        sid = lax.axis_index("s")
        wid = cid * NS + sid
        zero = jnp.zeros((16,), jnp.float32)

        def zj(j, carry):
            zb[pl.ds(j * 16, 16)] = zero
            return carry

        lax.fori_loop(0, NB // 16, zj, 0)
        pltpu.sync_copy(zb, acc_rs.at[pl.ds(sid * NB, NB)])
        pltpu.sync_copy(zb, acc_cnt.at[pl.ds(sid * NB, NB)])
        plsc.subcore_barrier()

        def chunk(k, carry):
            base = wid * EW + k * CH
            c1 = pltpu.async_copy(src_hbm.at[pl.ds(base, CH)], src_v, sem1)
            c2 = pltpu.async_copy(dst_hbm.at[pl.ds(base, CH)], dst_v, sem2)
            c1.wait()
            c2.wait()
            ca = pltpu.async_copy(xn_hbm.at[src_v], xs_v, sem1)
            cb = pltpu.async_copy(xn_hbm.at[dst_v], xd_v, sem2)
            ca.wait()
            cb.wait()

            lanes16 = lax.iota(jnp.int32, 16) * 16

            def grp(i, gcarry):
                for l in range(16):
                    e = i * 16 + l
                    acc = xs_v[e, pl.ds(0, 16)] * xd_v[e, pl.ds(0, 16)]
                    for f in range(1, F // 16):
                        sl = pl.ds(f * 16, 16)
                        acc = acc + xs_v[e, sl] * xd_v[e, sl]
                    red_v[pl.ds(l * 16, 16)] = acc
                raw = plsc.load_gather(red_v, [lanes16])
                for j in range(1, 16):
                    raw = raw + plsc.load_gather(red_v, [lanes16 + j])
                sl = pl.ds(i * 16, 16)
                s16 = src_v[sl]
                d16 = dst_v[sl]
                t16 = jnp.where((raw >= THRESH) & (s16 != d16), raw, 0.0)
                t_v[sl] = t16
                cnt_v[sl] = jnp.where(t16 > 0.0, 1.0, 0.0)
                return gcarry

            lax.fori_loop(0, CHV, grp, 0)
            pltpu.sync_copy(t_v, t_hbm.at[pl.ds(base, CH)])
            m = t_v[pl.ds(0, 16)]
            for i in range(1, CHV):
                m = jnp.maximum(m, t_v[pl.ds(i * 16, 16)])

            @pl.when(jnp.any(m > 0.0))
            def _do_adds():
                pltpu.sync_copy(t_v, acc_rs.at[src_v], add=True)
                pltpu.sync_copy(cnt_v, acc_cnt.at[src_v], add=True)

            return carry

        lax.fori_loop(0, EW // CH, chunk, 0)
        plsc.subcore_barrier()
        sl = pl.ds(sid * NB, NB)
        osl = pl.ds(cid * NP + sid * NB, NB)
        pltpu.sync_copy(acc_rs.at[sl], rs_hbm.at[osl])
        pltpu.sync_copy(acc_cnt.at[sl], cnt_hbm.at[osl])

    return p1


def _make_p2():
    mesh = _mesh()

    @functools.partial(
        pl.kernel, mesh=mesh,
        out_type=[
            jax.ShapeDtypeStruct((E,), jnp.float32),
            jax.ShapeDtypeStruct((NC * NP,), jnp.float32),
        ],
        scratch_types=[
            pltpu.VMEM((CH,), jnp.int32),
            pltpu.VMEM((CH,), jnp.int32),
            pltpu.VMEM((CH,), jnp.float32),
            pltpu.VMEM((CH,), jnp.float32),
            pltpu.VMEM((NP,), jnp.float32),
            pltpu.VMEM((NP,), jnp.float32),
            pltpu.VMEM((NB,), jnp.float32),
            pltpu.VMEM_SHARED((NP,), jnp.float32),
        ],
        compiler_params=pltpu.CompilerParams(needs_layout_passes=False),
        name="gnn_p2",
    )
    def p2(t_hbm, src_hbm, dst_hbm, rs_hbm, w_hbm, ws_hbm,
           src_v, dst_v, t_v, w_v, rs_full, tmp, zb, acc_ws):
        cid = lax.axis_index("c")
        sid = lax.axis_index("s")
        wid = cid * NS + sid
        zero = jnp.zeros((16,), jnp.float32)
        pltpu.sync_copy(rs_hbm.at[pl.ds(0, NP)], rs_full)
        pltpu.sync_copy(rs_hbm.at[pl.ds(NP, NP)], tmp)

        def addj(j, carry):
            sj = pl.ds(j * 16, 16)
            rs_full[sj] = rs_full[sj] + tmp[sj]
            return carry

        lax.fori_loop(0, NP // 16, addj, 0)

        def zj(j, carry):
            zb[pl.ds(j * 16, 16)] = zero
            return carry

        lax.fori_loop(0, NB // 16, zj, 0)
        pltpu.sync_copy(zb, acc_ws.at[pl.ds(sid * NB, NB)])
        plsc.subcore_barrier()

        def chunk(k, carry):
            base = wid * EW + k * CH
            pltpu.sync_copy(src_hbm.at[pl.ds(base, CH)], src_v)
            pltpu.sync_copy(dst_hbm.at[pl.ds(base, CH)], dst_v)
            pltpu.sync_copy(t_hbm.at[pl.ds(base, CH)], t_v)

            def grp(i, gcarry):
                sl = pl.ds(i * 16, 16)
                s16 = src_v[sl]
                rs16 = plsc.load_gather(rs_full, [s16])
                t16 = t_v[sl]
                w_v[sl] = jnp.where(rs16 > 0.0, t16 / rs16, 0.0)
                return gcarry

            lax.fori_loop(0, CHV, grp, 0)
            pltpu.sync_copy(w_v, w_hbm.at[pl.ds(base, CH)])
            pltpu.sync_copy(w_v, acc_ws.at[dst_v], add=True)
            return carry

        lax.fori_loop(0, EW // CH, chunk, 0)
        plsc.subcore_barrier()
        sl = pl.ds(sid * NB, NB)
        pltpu.sync_copy(acc_ws.at[sl],
                        ws_hbm.at[pl.ds(cid * NP + sid * NB, NB)])

    return p2


def _make_p3():
    mesh = _mesh()

    @functools.partial(
        pl.kernel, mesh=mesh,
        out_type=jax.ShapeDtypeStruct((NP, FH), jnp.float32),
        scratch_types=[
            pltpu.VMEM((CH,), jnp.int32),
            pltpu.VMEM((CH,), jnp.int32),
            pltpu.VMEM((CH,), jnp.float32),
            pltpu.VMEM((CH, FH), jnp.float32),
            pltpu.VMEM((NP,), jnp.float32),
            pltpu.VMEM((NP,), jnp.float32),
            pltpu.VMEM((ZR, FH), jnp.float32),
            pltpu.VMEM((ER, FH), jnp.float32),
            pltpu.VMEM((ER, FH), jnp.float32),
            pltpu.VMEM_SHARED((AR, FH), jnp.float32),
            pltpu.SemaphoreType.DMA,
        ],
        compiler_params=pltpu.CompilerParams(needs_layout_passes=False),
        name="gnn_p3",
    )
    def p3(h_hbm, w_hbm, src_hbm, dst_hbm, dinv_hbm, selfco_hbm, out_hbm,
           src_v, dst_v, c_v, hrow_v, dinv_t, selfco_t, zbuf, ebuf, hbuf,
           acc, sem1):
        cid = lax.axis_index("c")
        sid = lax.axis_index("s")
        lo = cid * NH
        zero16 = jnp.zeros((16,), jnp.float32)

        pltpu.sync_copy(dinv_hbm, dinv_t)
        pltpu.sync_copy(selfco_hbm, selfco_t)

        def zrow(r, carry):
            for f in range(FH // 16):
                zbuf[r, pl.ds(f * 16, 16)] = zero16
            return carry

        lax.fori_loop(0, ZR, zrow, 0)
        for z in range(AT // ZR):
            pltpu.sync_copy(zbuf, acc.at[pl.ds(sid * AT + z * ZR, ZR)])
        plsc.subcore_barrier()

        def chunk(k, carry):
            base = sid * NC * EW + k * CH
            pltpu.sync_copy(w_hbm.at[pl.ds(base, CH)], c_v)
            m = c_v[pl.ds(0, 16)]
            for i in range(1, CHV):
                m = jnp.maximum(m, c_v[pl.ds(i * 16, 16)])

            @pl.when(jnp.any(m > 0.0))
            def _do_chunk():
                c1 = pltpu.async_copy(src_hbm.at[pl.ds(base, CH)], src_v,
                                      sem1)
                pltpu.sync_copy(dst_hbm.at[pl.ds(base, CH)], dst_v)
                c1.wait()
                ca = pltpu.async_copy(h_hbm.at[src_v], hrow_v, sem1)

                def grp(i, gcarry):
                    sl = pl.ds(i * 16, 16)
                    s16 = src_v[sl]
                    d16 = dst_v[sl]
                    ds16 = plsc.load_gather(dinv_t, [s16])
                    dd16 = plsc.load_gather(dinv_t, [d16])
                    c_v[sl] = c_v[sl] * ds16 * dd16
                    dl16 = d16 - lo
                    mine = (dl16 >= 0) & (dl16 < NH)
                    dst_v[sl] = jnp.where(mine, dl16, NH)
                    return gcarry

                lax.fori_loop(0, CHV, grp, 0)
                ca.wait()

                def edge(e, ecarry):
                    cb = plsc.load_gather(
                        c_v, [jnp.full((16,), e, jnp.int32)])
                    for f in range(FH // 16):
                        sl = pl.ds(f * 16, 16)
                        hrow_v[e, sl] = hrow_v[e, sl] * cb
                    return ecarry

                lax.fori_loop(0, CH, edge, 0)
                pltpu.sync_copy(hrow_v, acc.at[dst_v], add=True)

            return carry

        lax.fori_loop(0, NC * EW // CH, chunk, 0)
        plsc.subcore_barrier()

        for chk in range(NT // ER):
            n0l = sid * NT + chk * ER
            n0g = lo + n0l
            pltpu.sync_copy(acc.at[pl.ds(n0l, ER)], ebuf)
            pltpu.sync_copy(h_hbm.at[pl.ds(n0g, ER)], hbuf)

            def srow(r, carry):
                cb = plsc.load_gather(
                    selfco_t, [jnp.full((16,), n0g + r, jnp.int32)])
                for f in range(FH // 16):
                    sl = pl.ds(f * 16, 16)
                    ebuf[r, sl] = ebuf[r, sl] + cb * hbuf[r, sl]
                return carry

            lax.fori_loop(0, ER, srow, 0)
            pltpu.sync_copy(ebuf, out_hbm.at[pl.ds(n0g, ER)])

    return p3


_P1_128 = _make_p1(128)
_P1_256 = _make_p1(256)
_P2 = _make_p2()
_P3 = _make_p3()


def _layer_sparse(xn, h_parts, src, dst, p1):
    t, rs, cnt = p1(xn, src, dst)
    w, ws = _P2(t, src, dst, rs)
    dinv, selfco = _tc_node(cnt, ws)
    out = []
    for h in h_parts:
        hp = jnp.pad(h, ((0, NP - N), (0, 0)))
        out.append(_P3(hp, w, src, dst, dinv, selfco)[:N])
    return out


def kernel(node_features, edge_index, edge_weight, W1, b1, W2, b2, W3, b3):
    src = edge_index[0]
    dst = edge_index[1]

    xn0, h0, h1 = _tc1(node_features, W1)
    y1a, y1b = _layer_sparse(xn0, [h0, h1], src, dst, _P1_128)

    xn1, h2 = _tc2(y1a, y1b, b1, W2)
    (y2,) = _layer_sparse(xn1, [h2], src, dst, _P1_256)

    W3p = jnp.pad(W3, ((0, 0), (0, 88)))
    xn2, h3 = _tc3(y2, b2, W3p)
    (y3,) = _layer_sparse(xn2, [h3], src, dst, _P1_128)

    return _tc4(y3, b3)
